# baseline (speedup 1.0000x reference)
import functools

import jax
import jax.numpy as jnp
from jax import lax
from jax.experimental import pallas as pl
from jax.experimental.pallas import tpu as pltpu
from jax.experimental.pallas import tpu_sc as plsc

F = 26
V = 100000
E = 16
B = 16384

NC, NS, L = 2, 16, 16
NW = NC * NS
J = F * E
RPW = J // NW
CHB = 2048
NCH = B // CHB
VA = (V // 128) * 128
VT = V - VA
TAILR = 112

_mesh = plsc.VectorSubcoreMesh(
    core_axis_name="c", subcore_axis_name="s", num_cores=NC, num_subcores=NS
)


@functools.partial(
    pl.kernel,
    out_type=jax.ShapeDtypeStruct((J, B), jnp.float32),
    mesh=_mesh,
    scratch_types=[
        pltpu.VMEM((1, V), jnp.float32),
        pltpu.VMEM((16, 128), jnp.float32),
        pltpu.VMEM((1, B), jnp.int32),
        pltpu.VMEM((2, 1, CHB), jnp.float32),
        pltpu.VMEM((1, 16), jnp.int32),
        pltpu.VMEM((1, 16), jnp.int32),
        pltpu.SemaphoreType.DMA,
        pltpu.SemaphoreType.DMA,
        pltpu.SemaphoreType.DMA,
        pltpu.SemaphoreType.DMA,
        pltpu.SemaphoreType.DMA,
        pltpu.SemaphoreType.DMA,
        pltpu.SemaphoreType.DMA,
    ],
    compiler_params=pltpu.CompilerParams(needs_layout_passes=False),
)
def _emb_kernel(
    xb_hbm, tbl_hbm, tail_hbm, out_hbm,
    erow_v, tail_v, xrow_v, oc_v, jv, fiv,
    tsem0, tsem1, tsem2, tsem3, isem, osem0, osem1,
):
    w = lax.axis_index("s") * NC + lax.axis_index("c")
    j0 = w * RPW

    zeros16 = jnp.zeros((L,), jnp.int32)
    iota16 = lax.iota(jnp.int32, L)

    r8 = lax.bitwise_and((j0 * VT) // 128, jnp.int32(~7))
    r8 = pl.multiple_of(r8, 8)
    pltpu.sync_copy(tail_hbm.at[pl.ds(r8, 16)], tail_v)

    osems = (osem0, osem1)

    def out_desc(slot, j_dyn, c_dyn):
        return pltpu.make_async_copy(
            oc_v.at[slot],
            out_hbm.at[jv.at[0, pl.ds(0, 1)], pl.ds(c_dyn * CHB, CHB)],
            osems[slot],
        )

    for r in range(RPW):
        j = j0 + r
        f = j // 16
        jv[0, :] = jnp.full((L,), j, jnp.int32)
        erow_cps = [
            pltpu.make_async_copy(
                tbl_hbm.at[jv.at[0, pl.ds(0, 1)], pl.ds(0, VA)],
                erow_v.at[:, pl.ds(0, VA)],
                tsem0,
            )
        ]
        for cp in erow_cps:
            cp.start()

        def stage_xrow():
            fiv[0, :] = jnp.full((L,), f, jnp.int32)
            pltpu.async_copy(
                xb_hbm.at[fiv.at[0, pl.ds(0, 1)]], xrow_v, isem
            ).wait()

        if r == 0:
            stage_xrow()
        else:
            pl.when(j % 16 == 0)(stage_xrow)

        for cp in erow_cps:
            cp.wait()
        flat0 = j * VT + iota16
        flat1 = flat0 + L
        erow_v[0, pl.ds(VA, L)] = plsc.load_gather(
            tail_v, [(flat0 >> 7) - r8, lax.bitwise_and(flat0, 127)]
        )
        erow_v[0, pl.ds(VA + L, L)] = plsc.load_gather(
            tail_v, [(flat1 >> 7) - r8, lax.bitwise_and(flat1, 127)]
        )

        def c2_body(c2, _):
            for s in range(2):
                c = c2 * 2 + s
                pl.when(jnp.logical_or(c2 > 0, r > 0))(
                    lambda: out_desc(s, j, c).wait()
                )

                @plsc.parallel_loop(0, CHB // L, unroll=16)
                def _gather_chunk(i):
                    vvec = xrow_v[0, pl.ds(c * CHB + i * L, L)]
                    oc_v[s, 0, pl.ds(i * L, L)] = plsc.load_gather(
                        erow_v, [zeros16, vvec]
                    )
                out_desc(s, j, c).start()
            return 0

        lax.fori_loop(0, NCH // 2, c2_body, 0)

    jlast = j0 + RPW - 1
    out_desc(0, jlast, NCH - 2).wait()
    out_desc(1, jlast, NCH - 1).wait()


def kernel(x, tables):
    tbl2d = jnp.transpose(tables, (0, 2, 1)).reshape(J, V)
    tail = jnp.transpose(tables[:, VA:, :], (0, 2, 1)).reshape(J * VT // 128, 128)
    tail = jnp.pad(tail, ((0, TAILR - J * VT // 128), (0, 0)))
    xb = jnp.transpose(x)
    out2d = _emb_kernel(xb, tbl2d, tail)
    return jnp.transpose(out2d)


# speedup vs baseline: 1.0163x; 1.0163x over previous
import functools

import jax
import jax.numpy as jnp
from jax import lax
from jax.experimental import pallas as pl
from jax.experimental.pallas import tpu as pltpu
from jax.experimental.pallas import tpu_sc as plsc

F = 26
V = 100000
E = 16
B = 16384

NC, NS, L = 2, 16, 16
NW = NC * NS
J = F * E
RPW = J // NW
CHB = 2048
NCH = B // CHB
VA = (V // 128) * 128
VT = V - VA
TAILR = 112

_mesh = plsc.VectorSubcoreMesh(
    core_axis_name="c", subcore_axis_name="s", num_cores=NC, num_subcores=NS
)


@functools.partial(
    pl.kernel,
    out_type=jax.ShapeDtypeStruct((J, B), jnp.float32),
    mesh=_mesh,
    scratch_types=[
        pltpu.VMEM((1, V), jnp.float32),
        pltpu.VMEM((16, 128), jnp.float32),
        pltpu.VMEM((1, B), jnp.int32),
        pltpu.VMEM((2, 1, CHB), jnp.float32),
        pltpu.VMEM((1, 16), jnp.int32),
        pltpu.VMEM((1, 16), jnp.int32),
        pltpu.SemaphoreType.DMA,
        pltpu.SemaphoreType.DMA,
        pltpu.SemaphoreType.DMA,
        pltpu.SemaphoreType.DMA,
        pltpu.SemaphoreType.DMA,
        pltpu.SemaphoreType.DMA,
        pltpu.SemaphoreType.DMA,
    ],
    compiler_params=pltpu.CompilerParams(needs_layout_passes=False),
)
def _emb_kernel(
    xb_hbm, tbl_hbm, tail_hbm, out_hbm,
    erow_v, tail_v, xrow_v, oc_v, jv, fiv,
    tsem0, tsem1, tsem2, tsem3, isem, osem0, osem1,
):
    w = lax.axis_index("s") * NC + lax.axis_index("c")
    j0 = w * RPW

    zeros16 = jnp.zeros((L,), jnp.int32)
    iota16 = lax.iota(jnp.int32, L)

    r8 = lax.bitwise_and((j0 * VT) // 128, jnp.int32(~7))
    r8 = pl.multiple_of(r8, 8)
    pltpu.sync_copy(tail_hbm.at[pl.ds(r8, 16)], tail_v)

    osems = (osem0, osem1)

    def out_desc(slot, j_dyn, c_dyn):
        return pltpu.make_async_copy(
            oc_v.at[slot],
            out_hbm.at[jv.at[0, pl.ds(0, 1)], pl.ds(c_dyn * CHB, CHB)],
            osems[slot],
        )

    for r in range(RPW):
        j = j0 + r
        f = j // 16
        jv[0, :] = jnp.full((L,), j, jnp.int32)
        erow_cps = [
            pltpu.make_async_copy(
                tbl_hbm.at[jv.at[0, pl.ds(0, 1)], pl.ds(0, VA)],
                erow_v.at[:, pl.ds(0, VA)],
                tsem0,
            )
        ]
        for cp in erow_cps:
            cp.start()

        def stage_xrow():
            fiv[0, :] = jnp.full((L,), f, jnp.int32)
            pltpu.async_copy(
                xb_hbm.at[fiv.at[0, pl.ds(0, 1)]], xrow_v, isem
            ).wait()

        if r == 0:
            stage_xrow()
        else:
            pl.when(j % 16 == 0)(stage_xrow)

        for cp in erow_cps:
            cp.wait()
        flat0 = j * VT + iota16
        flat1 = flat0 + L
        erow_v[0, pl.ds(VA, L)] = plsc.load_gather(
            tail_v, [(flat0 >> 7) - r8, lax.bitwise_and(flat0, 127)]
        )
        erow_v[0, pl.ds(VA + L, L)] = plsc.load_gather(
            tail_v, [(flat1 >> 7) - r8, lax.bitwise_and(flat1, 127)]
        )

        def c2_body(c2, _):
            for s in range(2):
                c = c2 * 2 + s
                pl.when(jnp.logical_or(c2 > 0, r > 0))(
                    lambda: out_desc(s, j, c).wait()
                )

                @plsc.parallel_loop(0, CHB // L, unroll=8)
                def _gather_chunk(i):
                    vvec = xrow_v[0, pl.ds(c * CHB + i * L, L)]
                    oc_v[s, 0, pl.ds(i * L, L)] = plsc.load_gather(
                        erow_v, [zeros16, vvec]
                    )
                out_desc(s, j, c).start()
            return 0

        lax.fori_loop(0, NCH // 2, c2_body, 0)

    jlast = j0 + RPW - 1
    out_desc(0, jlast, NCH - 2).wait()
    out_desc(1, jlast, NCH - 1).wait()


def kernel(x, tables):
    tbl2d = jnp.transpose(tables, (0, 2, 1)).reshape(J, V)
    tail = jnp.transpose(tables[:, VA:, :], (0, 2, 1)).reshape(J * VT // 128, 128)
    tail = jnp.pad(tail, ((0, TAILR - J * VT // 128), (0, 0)))
    xb = jnp.transpose(x)
    out2d = _emb_kernel(xb, tbl2d, tail)
    return jnp.transpose(out2d)
